# baseline (speedup 1.0000x reference)
import functools

import jax
import jax.numpy as jnp
from jax import lax
from jax.experimental import pallas as pl
from jax.experimental.pallas import tpu as pltpu
from jax.experimental.pallas import tpu_sc as plsc

T = 2048
H = 768
F = 3072
E = 8
K = 2
BLK = 128
FB = 768
NF = F // FB
NJ = T // BLK
NBT = (T * K) // BLK + E
PADTOT = NBT * BLK
NC = 2
NS = 16
NW = NC * NS
_INTERPRET = False


def _route_body(x_ref, gw_ref, dest_ref, gates_ref, nblk_ref, pbase_ref,
                loss_ref):
    x = x_ref[...]
    gw = gw_ref[...]
    logits = lax.dot_general(x.astype(jnp.bfloat16), gw.astype(jnp.bfloat16),
                             (((1,), (1,)), ((), ())),
                             preferred_element_type=jnp.float32)
    m = jnp.max(logits, axis=1, keepdims=True)
    ex = jnp.exp(logits - m)
    p = ex / jnp.sum(ex, axis=1, keepdims=True)

    iota_e = lax.broadcasted_iota(jnp.int32, (T, E), 1)
    m1 = jnp.max(p, axis=1, keepdims=True)
    i1 = jnp.min(jnp.where(p == m1, iota_e, E + 1), axis=1, keepdims=True)
    oh1 = iota_e == i1
    pm = jnp.where(oh1, -1.0, p)
    m2 = jnp.max(pm, axis=1, keepdims=True)
    i2 = jnp.min(jnp.where(pm == m2, iota_e, E + 1), axis=1, keepdims=True)
    oh2 = iota_e == i2

    denom = m1 + m2 + 1e-8
    g1 = m1 / denom
    g2 = m2 / denom

    oh1f = oh1.astype(jnp.float32)
    oh2f = oh2.astype(jnp.float32)
    counts = jnp.sum(oh1f + oh2f, axis=0, keepdims=True)

    total = jnp.sum(counts, axis=1, keepdims=True)
    usage = counts / total
    mean_u = jnp.sum(usage, axis=1, keepdims=True) / E
    var_u = jnp.sum((usage - mean_u) ** 2, axis=1, keepdims=True) / (E - 1)
    cv2 = (var_u / (mean_u + 1e-8)) ** 2
    loss_ref[...] = cv2

    nblk_f = jnp.floor((counts + (BLK - 1)) * (1.0 / BLK))
    tri8 = (lax.broadcasted_iota(jnp.int32, (E, E), 0)
            < lax.broadcasted_iota(jnp.int32, (E, E), 1)).astype(jnp.float32)
    pbase_f = lax.dot_general(nblk_f, tri8, (((1,), (0,)), ((), ())),
                              preferred_element_type=jnp.float32)
    nblk_ref[...] = nblk_f.astype(jnp.int32)
    pbase_ref[...] = pbase_f.astype(jnp.int32)

    M = jnp.concatenate([oh1f, oh2f], axis=0)
    CH = 512
    tri = (lax.broadcasted_iota(jnp.int32, (CH, CH), 1)
           < lax.broadcasted_iota(jnp.int32, (CH, CH), 0)).astype(jnp.float32)
    ranks = []
    carry = jnp.zeros((1, E), jnp.float32)
    for c in range((K * T) // CH):
        blkm = lax.slice(M, (c * CH, 0), (c * CH + CH, E))
        r = lax.dot_general(tri, blkm, (((1,), (0,)), ((), ())),
                            preferred_element_type=jnp.float32) + carry
        ranks.append(r)
        carry = carry + jnp.sum(blkm, axis=0, keepdims=True)
    ranks = jnp.concatenate(ranks, axis=0)

    ohP = jnp.concatenate([oh1f, oh2f], axis=0)
    rank_sel = jnp.sum(ranks * ohP, axis=1, keepdims=True)
    pb_sel = jnp.sum(ohP * pbase_f, axis=1, keepdims=True)
    dest_ref[...] = (pb_sel * BLK + rank_sel).astype(jnp.int32)
    gates_ref[...] = jnp.concatenate([g1, g2], axis=0)


def _route(x2, gate_w):
    return pl.pallas_call(
        _route_body,
        out_shape=(
            jax.ShapeDtypeStruct((K * T, 1), jnp.int32),
            jax.ShapeDtypeStruct((K * T, 1), jnp.float32),
            jax.ShapeDtypeStruct((1, E), jnp.int32),
            jax.ShapeDtypeStruct((1, E), jnp.int32),
            jax.ShapeDtypeStruct((1, 1), jnp.float32),
        ),
        interpret=_INTERPRET,
    )(x2, gate_w)


_RPW = T // NW


def _dispatch_body(dest_hbm, x_hbm, xs_hbm, idx0_v, idx1_v, rows_v, sem0, sem1):
    c = lax.axis_index("c")
    s = lax.axis_index("s")
    wid = s * NC + c
    base = wid * _RPW
    pltpu.sync_copy(x_hbm.at[pl.ds(base, _RPW)], rows_v)
    pltpu.sync_copy(dest_hbm.at[pl.ds(base, _RPW)], idx0_v)
    pltpu.sync_copy(dest_hbm.at[pl.ds(T + base, _RPW)], idx1_v)
    cp0 = pltpu.async_copy(rows_v, xs_hbm.at[idx0_v], sem0)
    cp1 = pltpu.async_copy(rows_v, xs_hbm.at[idx1_v], sem1)
    cp0.wait()
    cp1.wait()


def _dispatch_rows(dest1d, x2):
    mesh = plsc.VectorSubcoreMesh(core_axis_name="c", subcore_axis_name="s", num_cores=NC, num_subcores=NS)
    return pl.kernel(
        _dispatch_body,
        out_type=jax.ShapeDtypeStruct((PADTOT, H), jnp.float32),
        mesh=mesh,
        scratch_types=[
            pltpu.VMEM((_RPW,), jnp.int32),
            pltpu.VMEM((_RPW,), jnp.int32),
            pltpu.VMEM((_RPW, H), jnp.float32),
            pltpu.SemaphoreType.DMA,
            pltpu.SemaphoreType.DMA,
        ],
        compiler_params=pltpu.CompilerParams(needs_layout_passes=False),
        interpret=_INTERPRET,
    )(dest1d, x2)


def _ffn_body(nblk_ref, pbase_ref, xs_hbm, wg_ref, wu_ref, wd_ref, y_hbm,
              xs_v, y_v, sem_in, sem_out):
    e = pl.program_id(0)
    f = pl.program_id(1)

    @pl.when((e == 0) & (f == 0))
    def _():
        pltpu.make_async_copy(xs_hbm, xs_v, sem_in).start()
        pltpu.make_async_copy(xs_hbm, xs_v, sem_in).wait()

    nb = nblk_ref[e]
    base_blk = pbase_ref[e]
    wg = wg_ref[0].astype(jnp.bfloat16)
    wu = wu_ref[0].astype(jnp.bfloat16)
    wd = wd_ref[0].astype(jnp.bfloat16)
    first = f == 0

    def blk(j, _):
        row0 = pl.multiple_of((base_blk + j) * BLK, BLK)
        xb = xs_v[pl.ds(row0, BLK), :].astype(jnp.bfloat16)
        gproj = lax.dot_general(xb, wg, (((1,), (1,)), ((), ())),
                                preferred_element_type=jnp.float32)
        uproj = lax.dot_general(xb, wu, (((1,), (1,)), ((), ())),
                                preferred_element_type=jnp.float32)
        sig = 1.0 / (1.0 + jnp.exp(-uproj))
        hid = (gproj * (uproj * sig)).astype(jnp.bfloat16)
        part = lax.dot_general(hid, wd, (((1,), (1,)), ((), ())),
                               preferred_element_type=jnp.float32)
        prev = jnp.where(first, 0.0, y_v[pl.ds(row0, BLK), :])
        y_v[pl.ds(row0, BLK), :] = part + prev
        return 0

    lax.fori_loop(0, nb, blk, 0)

    @pl.when((e == E - 1) & (f == NF - 1))
    def _():
        pltpu.make_async_copy(y_v, y_hbm, sem_out).start()
        pltpu.make_async_copy(y_v, y_hbm, sem_out).wait()


def _ffn(nblk, pbase, xs, Wg, Wu, Wd):
    return pl.pallas_call(
        _ffn_body,
        grid=(E, NF),
        in_specs=[
            pl.BlockSpec(memory_space=pltpu.SMEM),
            pl.BlockSpec(memory_space=pltpu.SMEM),
            pl.BlockSpec(memory_space=pl.ANY),
            pl.BlockSpec((1, FB, H), lambda e, f: (e, f, 0)),
            pl.BlockSpec((1, FB, H), lambda e, f: (e, f, 0)),
            pl.BlockSpec((1, H, FB), lambda e, f: (e, 0, f)),
        ],
        out_specs=pl.BlockSpec(memory_space=pl.ANY),
        out_shape=jax.ShapeDtypeStruct((PADTOT, H), jnp.float32),
        scratch_shapes=[
            pltpu.VMEM((PADTOT, H), jnp.float32),
            pltpu.VMEM((PADTOT, H), jnp.float32),
            pltpu.SemaphoreType.DMA,
            pltpu.SemaphoreType.DMA,
        ],
        interpret=_INTERPRET,
    )(nblk, pbase, xs, Wg, Wu, Wd)


def _combine_body(dest_hbm, gates_hbm, y_hbm, out_hbm,
                  idx0_v, idx1_v, g0_v, g1_v, r0_v, r1_v, sem0, sem1):
    c = lax.axis_index("c")
    s = lax.axis_index("s")
    wid = s * NC + c
    base = wid * _RPW
    pltpu.sync_copy(dest_hbm.at[pl.ds(base, _RPW)], idx0_v)
    pltpu.sync_copy(dest_hbm.at[pl.ds(T + base, _RPW)], idx1_v)
    pltpu.sync_copy(gates_hbm.at[pl.ds(base, _RPW)], g0_v)
    pltpu.sync_copy(gates_hbm.at[pl.ds(T + base, _RPW)], g1_v)
    cp0 = pltpu.async_copy(y_hbm.at[idx0_v], r0_v, sem0)
    cp1 = pltpu.async_copy(y_hbm.at[idx1_v], r1_v, sem1)
    cp0.wait()
    cp1.wait()

    def row(i, _):
        gi0 = plsc.load_gather(g0_v, [jnp.full((16,), i, jnp.int32)])
        gi1 = plsc.load_gather(g1_v, [jnp.full((16,), i, jnp.int32)])
        for jc in range(H // 16):
            r0_v[i, pl.ds(jc * 16, 16)] = (
                gi0 * r0_v[i, pl.ds(jc * 16, 16)]
                + gi1 * r1_v[i, pl.ds(jc * 16, 16)])
        return 0

    lax.fori_loop(0, _RPW, row, 0)
    pltpu.sync_copy(r0_v, out_hbm.at[pl.ds(base, _RPW)])


def _combine(dest1d, gates1d, y):
    mesh = plsc.VectorSubcoreMesh(core_axis_name="c", subcore_axis_name="s", num_cores=NC, num_subcores=NS)
    return pl.kernel(
        _combine_body,
        out_type=jax.ShapeDtypeStruct((T, H), jnp.float32),
        mesh=mesh,
        scratch_types=[
            pltpu.VMEM((_RPW,), jnp.int32),
            pltpu.VMEM((_RPW,), jnp.int32),
            pltpu.VMEM((_RPW,), jnp.float32),
            pltpu.VMEM((_RPW,), jnp.float32),
            pltpu.VMEM((_RPW, H), jnp.float32),
            pltpu.VMEM((_RPW, H), jnp.float32),
            pltpu.SemaphoreType.DMA,
            pltpu.SemaphoreType.DMA,
        ],
        compiler_params=pltpu.CompilerParams(needs_layout_passes=False),
        interpret=_INTERPRET,
    )(dest1d, gates1d, y)


def kernel(x, gate_w, Wg, Wu, Wd):
    B, S, _ = x.shape
    x2 = x.reshape(T, H)
    dest, gates, nblk, pbase, loss = _route(x2, gate_w)
    dest1d = dest.reshape(K * T)
    gates1d = gates.reshape(K * T)
    xs = _dispatch_rows(dest1d, x2)
    y = _ffn(nblk.reshape(E), pbase.reshape(E), xs, Wg, Wu, Wd)
    out = _combine(dest1d, gates1d, y)
    return (out.reshape(B, S, H), loss.reshape(()))


# speedup vs baseline: 1.5620x; 1.5620x over previous
import functools

import jax
import jax.numpy as jnp
from jax import lax
from jax.experimental import pallas as pl
from jax.experimental.pallas import tpu as pltpu
from jax.experimental.pallas import tpu_sc as plsc

T = 2048
H = 768
F = 3072
E = 8
K = 2
BLK = 256
FB = 768
NF = F // FB
NJ = T // BLK
NBT = (T * K) // BLK + E
PADTOT = NBT * BLK
NC = 2
NS = 16
NW = NC * NS


def _route_body(x_ref, gw_ref, dest_ref, gates_ref, nblk_ref, pbase_ref,
                loss_ref):
    x = x_ref[...]
    gw = gw_ref[...]
    logits = lax.dot_general(x.astype(jnp.bfloat16), gw.astype(jnp.bfloat16),
                             (((1,), (1,)), ((), ())),
                             preferred_element_type=jnp.float32)
    m = jnp.max(logits, axis=1, keepdims=True)
    ex = jnp.exp(logits - m)
    p = ex / jnp.sum(ex, axis=1, keepdims=True)

    iota_e = lax.broadcasted_iota(jnp.int32, (T, E), 1)
    m1 = jnp.max(p, axis=1, keepdims=True)
    i1 = jnp.min(jnp.where(p == m1, iota_e, E + 1), axis=1, keepdims=True)
    oh1 = iota_e == i1
    pm = jnp.where(oh1, -1.0, p)
    m2 = jnp.max(pm, axis=1, keepdims=True)
    i2 = jnp.min(jnp.where(pm == m2, iota_e, E + 1), axis=1, keepdims=True)
    oh2 = iota_e == i2

    denom = m1 + m2 + 1e-8
    g1 = m1 / denom
    g2 = m2 / denom

    oh1f = oh1.astype(jnp.float32)
    oh2f = oh2.astype(jnp.float32)
    counts = jnp.sum(oh1f + oh2f, axis=0, keepdims=True)

    total = jnp.sum(counts, axis=1, keepdims=True)
    usage = counts / total
    mean_u = jnp.sum(usage, axis=1, keepdims=True) / E
    var_u = jnp.sum((usage - mean_u) ** 2, axis=1, keepdims=True) / (E - 1)
    cv2 = (var_u / (mean_u + 1e-8)) ** 2
    loss_ref[...] = cv2

    nblk_f = jnp.floor((counts + (BLK - 1)) * (1.0 / BLK))
    tri8 = (lax.broadcasted_iota(jnp.int32, (E, E), 0)
            < lax.broadcasted_iota(jnp.int32, (E, E), 1)).astype(jnp.float32)
    pbase_f = lax.dot_general(nblk_f, tri8, (((1,), (0,)), ((), ())),
                              preferred_element_type=jnp.float32)
    nblk_ref[...] = nblk_f.astype(jnp.int32)
    pbase_ref[...] = pbase_f.astype(jnp.int32)

    M = jnp.concatenate([oh1f, oh2f], axis=0)
    CH = 512
    tri = (lax.broadcasted_iota(jnp.int32, (CH, CH), 1)
           < lax.broadcasted_iota(jnp.int32, (CH, CH), 0)).astype(jnp.float32)
    ranks = []
    carry = jnp.zeros((1, E), jnp.float32)
    for c in range((K * T) // CH):
        blkm = lax.slice(M, (c * CH, 0), (c * CH + CH, E))
        r = lax.dot_general(tri, blkm, (((1,), (0,)), ((), ())),
                            preferred_element_type=jnp.float32) + carry
        ranks.append(r)
        carry = carry + jnp.sum(blkm, axis=0, keepdims=True)
    ranks = jnp.concatenate(ranks, axis=0)

    ohP = jnp.concatenate([oh1f, oh2f], axis=0)
    rank_sel = jnp.sum(ranks * ohP, axis=1, keepdims=True)
    pb_sel = jnp.sum(ohP * pbase_f, axis=1, keepdims=True)
    dest_ref[...] = (pb_sel * BLK + rank_sel).astype(jnp.int32)
    gates_ref[...] = jnp.concatenate([g1, g2], axis=0)


def _route(x2, gate_w):
    return pl.pallas_call(
        _route_body,
        out_shape=(
            jax.ShapeDtypeStruct((K * T, 1), jnp.int32),
            jax.ShapeDtypeStruct((K * T, 1), jnp.float32),
            jax.ShapeDtypeStruct((1, E), jnp.int32),
            jax.ShapeDtypeStruct((1, E), jnp.int32),
            jax.ShapeDtypeStruct((1, 1), jnp.float32),
        ),
    )(x2, gate_w)


_RPW = T // NW


def _dispatch_body(dest_hbm, x_hbm, xs_hbm, idx0_v, idx1_v, rows_v, sem0, sem1):
    c = lax.axis_index("c")
    s = lax.axis_index("s")
    wid = s * NC + c
    base = wid * _RPW
    pltpu.sync_copy(x_hbm.at[pl.ds(base, _RPW)], rows_v)
    pltpu.sync_copy(dest_hbm.at[pl.ds(base, _RPW)], idx0_v)
    pltpu.sync_copy(dest_hbm.at[pl.ds(T + base, _RPW)], idx1_v)
    cp0 = pltpu.async_copy(rows_v, xs_hbm.at[idx0_v], sem0)
    cp1 = pltpu.async_copy(rows_v, xs_hbm.at[idx1_v], sem1)
    cp0.wait()
    cp1.wait()


def _dispatch_rows(dest1d, x2):
    mesh = plsc.VectorSubcoreMesh(core_axis_name="c", subcore_axis_name="s", num_cores=NC, num_subcores=NS)
    return pl.kernel(
        _dispatch_body,
        out_type=jax.ShapeDtypeStruct((PADTOT, H), jnp.float32),
        mesh=mesh,
        scratch_types=[
            pltpu.VMEM((_RPW,), jnp.int32),
            pltpu.VMEM((_RPW,), jnp.int32),
            pltpu.VMEM((_RPW, H), jnp.float32),
            pltpu.SemaphoreType.DMA,
            pltpu.SemaphoreType.DMA,
        ],
        compiler_params=pltpu.CompilerParams(needs_layout_passes=False),
    )(dest1d, x2)


def _ffn_body(nblk_ref, pbase_ref, xs_hbm, wg_ref, wu_ref, wd_ref, y_hbm,
              xs_v, y_v, sem_in, sem_out):
    e = pl.program_id(0)
    f = pl.program_id(1)

    @pl.when((e == 0) & (f == 0))
    def _():
        pltpu.make_async_copy(xs_hbm, xs_v, sem_in).start()
        pltpu.make_async_copy(xs_hbm, xs_v, sem_in).wait()

    nb = nblk_ref[e]
    base_blk = pbase_ref[e]
    wg = wg_ref[0].astype(jnp.bfloat16)
    wu = wu_ref[0].astype(jnp.bfloat16)
    wd = wd_ref[0].astype(jnp.bfloat16)
    first = f == 0

    def blk(j, _):
        row0 = pl.multiple_of((base_blk + j) * BLK, BLK)
        xb = xs_v[pl.ds(row0, BLK), :].astype(jnp.bfloat16)
        gproj = lax.dot_general(xb, wg, (((1,), (1,)), ((), ())),
                                preferred_element_type=jnp.float32)
        uproj = lax.dot_general(xb, wu, (((1,), (1,)), ((), ())),
                                preferred_element_type=jnp.float32)
        sig = 1.0 / (1.0 + jnp.exp(-uproj))
        hid = (gproj * (uproj * sig)).astype(jnp.bfloat16)
        part = lax.dot_general(hid, wd, (((1,), (1,)), ((), ())),
                               preferred_element_type=jnp.float32)
        prev = jnp.where(first, 0.0, y_v[pl.ds(row0, BLK), :])
        y_v[pl.ds(row0, BLK), :] = part + prev
        return 0

    lax.fori_loop(0, nb, blk, 0)

    @pl.when((e == E - 1) & (f == NF - 1))
    def _():
        pltpu.make_async_copy(y_v, y_hbm, sem_out).start()
        pltpu.make_async_copy(y_v, y_hbm, sem_out).wait()


def _ffn(nblk, pbase, xs, Wg, Wu, Wd):
    return pl.pallas_call(
        _ffn_body,
        grid=(E, NF),
        in_specs=[
            pl.BlockSpec(memory_space=pltpu.SMEM),
            pl.BlockSpec(memory_space=pltpu.SMEM),
            pl.BlockSpec(memory_space=pl.ANY),
            pl.BlockSpec((1, FB, H), lambda e, f: (e, f, 0)),
            pl.BlockSpec((1, FB, H), lambda e, f: (e, f, 0)),
            pl.BlockSpec((1, H, FB), lambda e, f: (e, 0, f)),
        ],
        out_specs=pl.BlockSpec(memory_space=pl.ANY),
        out_shape=jax.ShapeDtypeStruct((PADTOT, H), jnp.float32),
        scratch_shapes=[
            pltpu.VMEM((PADTOT, H), jnp.float32),
            pltpu.VMEM((PADTOT, H), jnp.float32),
            pltpu.SemaphoreType.DMA,
            pltpu.SemaphoreType.DMA,
        ],
    )(nblk, pbase, xs, Wg, Wu, Wd)


def _combine_body(dest_hbm, gates_hbm, y_hbm, out_hbm,
                  idx0_v, idx1_v, g0_v, g1_v, r0_v, r1_v, sem0, sem1):
    c = lax.axis_index("c")
    s = lax.axis_index("s")
    wid = s * NC + c
    base = wid * _RPW
    pltpu.sync_copy(dest_hbm.at[pl.ds(base, _RPW)], idx0_v)
    pltpu.sync_copy(dest_hbm.at[pl.ds(T + base, _RPW)], idx1_v)
    pltpu.sync_copy(gates_hbm.at[pl.ds(base, _RPW)], g0_v)
    pltpu.sync_copy(gates_hbm.at[pl.ds(T + base, _RPW)], g1_v)
    cp0 = pltpu.async_copy(y_hbm.at[idx0_v], r0_v, sem0)
    cp1 = pltpu.async_copy(y_hbm.at[idx1_v], r1_v, sem1)
    cp0.wait()
    cp1.wait()

    def row(i, _):
        gi0 = plsc.load_gather(g0_v, [jnp.full((16,), i, jnp.int32)])
        gi1 = plsc.load_gather(g1_v, [jnp.full((16,), i, jnp.int32)])
        for jc in range(H // 16):
            r0_v[i, pl.ds(jc * 16, 16)] = (
                gi0 * r0_v[i, pl.ds(jc * 16, 16)]
                + gi1 * r1_v[i, pl.ds(jc * 16, 16)])
        return 0

    lax.fori_loop(0, _RPW, row, 0)
    pltpu.sync_copy(r0_v, out_hbm.at[pl.ds(base, _RPW)])


def _combine(dest1d, gates1d, y):
    mesh = plsc.VectorSubcoreMesh(core_axis_name="c", subcore_axis_name="s", num_cores=NC, num_subcores=NS)
    return pl.kernel(
        _combine_body,
        out_type=jax.ShapeDtypeStruct((T, H), jnp.float32),
        mesh=mesh,
        scratch_types=[
            pltpu.VMEM((_RPW,), jnp.int32),
            pltpu.VMEM((_RPW,), jnp.int32),
            pltpu.VMEM((_RPW,), jnp.float32),
            pltpu.VMEM((_RPW,), jnp.float32),
            pltpu.VMEM((_RPW, H), jnp.float32),
            pltpu.VMEM((_RPW, H), jnp.float32),
            pltpu.SemaphoreType.DMA,
            pltpu.SemaphoreType.DMA,
        ],
        compiler_params=pltpu.CompilerParams(needs_layout_passes=False),
    )(dest1d, gates1d, y)


def kernel(x, gate_w, Wg, Wu, Wd):
    B, S, _ = x.shape
    x2 = x.reshape(T, H)
    dest, gates, nblk, pbase, loss = _route(x2, gate_w)
    dest1d = dest.reshape(K * T)
    gates1d = gates.reshape(K * T)
    xs = _dispatch_rows(dest1d, x2)
    y = _ffn(nblk.reshape(E), pbase.reshape(E), xs, Wg, Wu, Wd)
    out = _combine(dest1d, gates1d, y)
    return (out.reshape(B, S, H), loss.reshape(()))
